# Initial kernel scaffold; baseline (speedup 1.0000x reference)
import jax
import jax.numpy as jnp
from jax.experimental import pallas as pl


def kernel(x, edge_index, W_self0, W_neigh0, b0, W_self1, W_neigh1, b1, fc_W, fc_b, bn_gamma, bn_beta, W21, b21, W22, b22):
    raise NotImplementedError("write your pallas kernel here")


# speedup vs baseline: 4.7092x; 4.7092x over previous
import functools

import jax
import jax.numpy as jnp
from jax import lax
from jax.experimental import pallas as pl
from jax.experimental.pallas import tpu as pltpu
from jax.experimental.pallas import tpu_sc as plsc

N = 10000
E = 320000
D = 128

NC = 2
NS = 16
NW = NC * NS
EPW = E // NW
C = 80
NCHUNK = EPW // C
NPAD = 10240
RPW = NPAD // NS


def _sc_agg_kernel(h_hbm, src_hbm, dst_hbm, zero_hbm, agg_hbm,
                   src_v, dst_v, rows_v, agg_sh):
    cid = lax.axis_index("c")
    sid = lax.axis_index("s")
    wid = cid * NS + sid

    r0 = pl.multiple_of(sid * RPW, 8)
    pltpu.sync_copy(zero_hbm, agg_sh.at[pl.ds(r0, RPW)])
    plsc.subcore_barrier()

    base = wid * EPW

    def body(k, _):
        off = pl.multiple_of(base + k * C, 8)
        pltpu.sync_copy(src_hbm.at[pl.ds(off, C)], src_v)
        pltpu.sync_copy(dst_hbm.at[pl.ds(off, C)], dst_v)
        pltpu.sync_copy(h_hbm.at[src_v], rows_v)
        pltpu.sync_copy(rows_v, agg_sh.at[dst_v], add=True)
        return 0

    lax.fori_loop(0, NCHUNK, body, 0)

    plsc.subcore_barrier()
    pltpu.sync_copy(agg_sh.at[pl.ds(r0, RPW)], agg_hbm.at[cid, pl.ds(r0, RPW)])


def _sc_deg_kernel(dst_hbm, zero_hbm, ones_hbm, deg_hbm,
                   dst_v, ones_v, deg_sh):
    cid = lax.axis_index("c")
    sid = lax.axis_index("s")
    wid = cid * NS + sid

    r0 = pl.multiple_of(sid * RPW, 8)
    pltpu.sync_copy(zero_hbm, deg_sh.at[pl.ds(r0, RPW)])
    pltpu.sync_copy(ones_hbm, ones_v)
    plsc.subcore_barrier()

    base = wid * EPW

    def body(k, _):
        off = pl.multiple_of(base + k * C, 8)
        pltpu.sync_copy(dst_hbm.at[pl.ds(off, C)], dst_v)
        pltpu.sync_copy(ones_v, deg_sh.at[dst_v], add=True)
        return 0

    lax.fori_loop(0, NCHUNK, body, 0)

    plsc.subcore_barrier()
    pltpu.sync_copy(deg_sh.at[pl.ds(r0, RPW)], deg_hbm.at[cid, pl.ds(r0, RPW)])


@functools.cache
def _sc_calls():
    mesh = plsc.VectorSubcoreMesh(core_axis_name="c", subcore_axis_name="s",
                                  num_cores=NC, num_subcores=NS)
    agg = functools.partial(
        pl.kernel,
        out_type=jax.ShapeDtypeStruct((NC, NPAD, D), jnp.float32),
        mesh=mesh,
        scratch_types=[
            pltpu.VMEM((C,), jnp.int32),
            pltpu.VMEM((C,), jnp.int32),
            pltpu.VMEM((C, D), jnp.float32),
            pltpu.VMEM_SHARED((NPAD, D), jnp.float32),
        ],
    )(_sc_agg_kernel)
    deg = functools.partial(
        pl.kernel,
        out_type=jax.ShapeDtypeStruct((NC, NPAD, D), jnp.float32),
        mesh=mesh,
        scratch_types=[
            pltpu.VMEM((C,), jnp.int32),
            pltpu.VMEM((C, D), jnp.float32),
            pltpu.VMEM_SHARED((NPAD, D), jnp.float32),
        ],
    )(_sc_deg_kernel)
    return agg, deg


def _prep_body(x_ref, out_ref):
    out_ref[...] = jnp.log(x_ref[...] + 1.0)


def _layer0_body(g_ref, agg_ref, deg_ref, ws_ref, wn_ref, b_ref,
                 out_ref, inv_ref):
    inv = 1.0 / jnp.maximum(deg_ref[0, :N, 0:1] + deg_ref[1, :N, 0:1], 1.0)
    inv_ref[...] = inv
    hn = (agg_ref[0, :N] + agg_ref[1, :N]) * inv
    h = (jnp.dot(g_ref[...], ws_ref[...], preferred_element_type=jnp.float32)
         + jnp.dot(hn, wn_ref[...], preferred_element_type=jnp.float32)
         + b_ref[...])
    h = jnp.maximum(h, 0.0)
    nrm = jnp.sqrt(jnp.sum(h * h, axis=1, keepdims=True))
    out_ref[...] = h / jnp.maximum(nrm, 1e-12)


def _final_body(h_ref, agg_ref, inv_ref, ws_ref, wn_ref, b_ref, fcw_ref,
                fcb_ref, gam_ref, bet_ref, w21_ref, b21_ref, w22_ref, b22_ref,
                zl_ref, zs_ref):
    hn = (agg_ref[0, :N] + agg_ref[1, :N]) * inv_ref[...]
    h2 = (jnp.dot(h_ref[...], ws_ref[...], preferred_element_type=jnp.float32)
          + jnp.dot(hn, wn_ref[...], preferred_element_type=jnp.float32)
          + b_ref[...])
    t = jnp.dot(h2, fcw_ref[...], preferred_element_type=jnp.float32) + fcb_ref[...]
    mu = jnp.mean(t, axis=0, keepdims=True)
    var = jnp.mean((t - mu) ** 2, axis=0, keepdims=True)
    t = (t - mu) * lax.rsqrt(var + 1e-5) * gam_ref[...] + bet_ref[...]
    t = jnp.maximum(t, 0.0)
    t = jnp.log(1.0 + jnp.exp(-t)) + t
    zl_ref[...] = jnp.dot(t, w21_ref[...], preferred_element_type=jnp.float32) + b21_ref[...]
    zs_ref[...] = jnp.exp(
        jnp.dot(t, w22_ref[...], preferred_element_type=jnp.float32) + b22_ref[...])


_prep = pl.pallas_call(
    _prep_body, out_shape=jax.ShapeDtypeStruct((N, D), jnp.float32))

_layer0 = pl.pallas_call(
    _layer0_body,
    out_shape=(jax.ShapeDtypeStruct((N, D), jnp.float32),
               jax.ShapeDtypeStruct((N, 1), jnp.float32)))

_final = pl.pallas_call(
    _final_body,
    out_shape=(jax.ShapeDtypeStruct((N, D), jnp.float32),
               jax.ShapeDtypeStruct((N, D), jnp.float32)))


def kernel(x, edge_index, W_self0, W_neigh0, b0, W_self1, W_neigh1, b1,
           fc_W, fc_b, bn_gamma, bn_beta, W21, b21, W22, b22):
    src = edge_index[0]
    dst = edge_index[1]
    zero_block = jnp.zeros((RPW, D), jnp.float32)
    ones_block = jnp.ones((C, D), jnp.float32)

    sc_agg, sc_deg = _sc_calls()
    g = _prep(x)
    degf = sc_deg(dst, zero_block, ones_block)
    agg0 = sc_agg(g, src, dst, zero_block)
    h1, inv = _layer0(g, agg0, degf, W_self0, W_neigh0, b0)
    agg1 = sc_agg(h1, src, dst, zero_block)
    z_loc, z_scale = _final(h1, agg1, inv, W_self1, W_neigh1, b1,
                            fc_W, fc_b, bn_gamma, bn_beta, W21, b21, W22, b22)
    return z_loc, z_scale


# speedup vs baseline: 8.3157x; 1.7659x over previous
import functools

import jax
import jax.numpy as jnp
from jax import lax
from jax.experimental import pallas as pl
from jax.experimental.pallas import tpu as pltpu
from jax.experimental.pallas import tpu_sc as plsc

N = 10000
E = 320000
D = 128

NC = 2
NS = 16
NW = NC * NS
EPW = E // NW
C = 80
NCH = EPW // C
NPAD = 10240
RPW = NPAD // NS


def _sc_agg_kernel(h_hbm, src_hbm, dst_hbm, zero_hbm, agg_hbm,
                   src_vm, dst_vm, rows_v, agg_sh, gsem, ssem):
    cid = lax.axis_index("c")
    sid = lax.axis_index("s")
    wid = cid * NS + sid

    pltpu.sync_copy(src_hbm.at[pl.ds(pl.multiple_of(wid * EPW, 8), EPW)], src_vm)
    pltpu.sync_copy(dst_hbm.at[wid], dst_vm)
    r0 = pl.multiple_of(sid * RPW, 8)
    pltpu.sync_copy(zero_hbm, agg_sh.at[pl.ds(r0, RPW)])
    pltpu.async_copy(h_hbm.at[src_vm.at[pl.ds(0, C)]], rows_v.at[0], gsem.at[0])
    plsc.subcore_barrier()

    def body(k, _):
        bk = lax.rem(k, 2)
        bn = 1 - bk
        pltpu.make_async_copy(
            h_hbm.at[src_vm.at[pl.ds(pl.multiple_of(k * C, 8), C)]], rows_v.at[bk], gsem.at[bk]).wait()

        @pl.when(k >= 1)
        def _():
            pltpu.make_async_copy(
                rows_v.at[bn], agg_sh.at[dst_vm.at[k - 1]], ssem.at[bn]).wait()

        @pl.when(k < NCH - 1)
        def _():
            pltpu.async_copy(
                h_hbm.at[src_vm.at[pl.ds(pl.multiple_of((k + 1) * C, 8), C)]], rows_v.at[bn], gsem.at[bn])

        pltpu.async_copy(rows_v.at[bk], agg_sh.at[dst_vm.at[k]], ssem.at[bk],
                         add=True)
        return 0

    lax.fori_loop(0, NCH, body, 0)
    pltpu.make_async_copy(
        rows_v.at[(NCH - 1) % 2], agg_sh.at[dst_vm.at[NCH - 1]],
        ssem.at[(NCH - 1) % 2]).wait()

    plsc.subcore_barrier()
    pltpu.sync_copy(agg_sh.at[pl.ds(r0, RPW)], agg_hbm.at[cid, pl.ds(r0, RPW)])


def _sc_deg_kernel(dst_hbm, zero_hbm, ones_hbm, deg_hbm,
                   dst_vm, ones_v, deg_sh, ssem):
    cid = lax.axis_index("c")
    sid = lax.axis_index("s")
    wid = cid * NS + sid

    pltpu.sync_copy(dst_hbm.at[wid], dst_vm)
    r0 = pl.multiple_of(sid * RPW, 8)
    pltpu.sync_copy(zero_hbm, deg_sh.at[pl.ds(r0, RPW)])
    pltpu.sync_copy(ones_hbm, ones_v)
    plsc.subcore_barrier()

    def body(k, _):
        @pl.when(k >= 1)
        def _():
            pltpu.make_async_copy(
                ones_v, deg_sh.at[dst_vm.at[k - 1]],
                ssem.at[lax.rem(k - 1, 2)]).wait()

        pltpu.async_copy(ones_v, deg_sh.at[dst_vm.at[k]],
                         ssem.at[lax.rem(k, 2)], add=True)
        return 0

    lax.fori_loop(0, NCH, body, 0)
    pltpu.make_async_copy(
        ones_v, deg_sh.at[dst_vm.at[NCH - 1]], ssem.at[(NCH - 1) % 2]).wait()

    plsc.subcore_barrier()
    pltpu.sync_copy(deg_sh.at[pl.ds(r0, RPW)], deg_hbm.at[cid, pl.ds(r0, RPW)])


@functools.cache
def _sc_calls():
    mesh = plsc.VectorSubcoreMesh(core_axis_name="c", subcore_axis_name="s",
                                  num_cores=NC, num_subcores=NS)
    agg = functools.partial(
        pl.kernel,
        out_type=jax.ShapeDtypeStruct((NC, NPAD, D), jnp.float32),
        mesh=mesh,
        scratch_types=[
            pltpu.VMEM((EPW,), jnp.int32),
            pltpu.VMEM((NCH, C), jnp.int32),
            pltpu.VMEM((2, C, D), jnp.float32),
            pltpu.VMEM_SHARED((NPAD, D), jnp.float32),
            pltpu.SemaphoreType.DMA((2,)),
            pltpu.SemaphoreType.DMA((2,)),
        ],
    )(_sc_agg_kernel)
    deg = functools.partial(
        pl.kernel,
        out_type=jax.ShapeDtypeStruct((NC, NPAD, D), jnp.float32),
        mesh=mesh,
        scratch_types=[
            pltpu.VMEM((NCH, C), jnp.int32),
            pltpu.VMEM((C, D), jnp.float32),
            pltpu.VMEM_SHARED((NPAD, D), jnp.float32),
            pltpu.SemaphoreType.DMA((2,)),
        ],
    )(_sc_deg_kernel)
    return agg, deg


def _prep_body(x_ref, out_ref):
    out_ref[...] = jnp.log(x_ref[...] + 1.0)


def _layer0_body(g_ref, agg_ref, deg_ref, ws_ref, wn_ref, b_ref,
                 out_ref, inv_ref):
    inv = 1.0 / jnp.maximum(deg_ref[0, :N, 0:1] + deg_ref[1, :N, 0:1], 1.0)
    inv_ref[...] = inv
    hn = (agg_ref[0, :N] + agg_ref[1, :N]) * inv
    h = (jnp.dot(g_ref[...], ws_ref[...], preferred_element_type=jnp.float32)
         + jnp.dot(hn, wn_ref[...], preferred_element_type=jnp.float32)
         + b_ref[...])
    h = jnp.maximum(h, 0.0)
    nrm = jnp.sqrt(jnp.sum(h * h, axis=1, keepdims=True))
    out_ref[...] = h / jnp.maximum(nrm, 1e-12)


def _final_body(h_ref, agg_ref, inv_ref, ws_ref, wn_ref, b_ref, fcw_ref,
                fcb_ref, gam_ref, bet_ref, w21_ref, b21_ref, w22_ref, b22_ref,
                zl_ref, zs_ref):
    hn = (agg_ref[0, :N] + agg_ref[1, :N]) * inv_ref[...]
    h2 = (jnp.dot(h_ref[...], ws_ref[...], preferred_element_type=jnp.float32)
          + jnp.dot(hn, wn_ref[...], preferred_element_type=jnp.float32)
          + b_ref[...])
    t = jnp.dot(h2, fcw_ref[...], preferred_element_type=jnp.float32) + fcb_ref[...]
    mu = jnp.mean(t, axis=0, keepdims=True)
    var = jnp.mean((t - mu) ** 2, axis=0, keepdims=True)
    t = (t - mu) * lax.rsqrt(var + 1e-5) * gam_ref[...] + bet_ref[...]
    t = jnp.maximum(t, 0.0)
    t = jnp.log(1.0 + jnp.exp(-t)) + t
    zl_ref[...] = jnp.dot(t, w21_ref[...], preferred_element_type=jnp.float32) + b21_ref[...]
    zs_ref[...] = jnp.exp(
        jnp.dot(t, w22_ref[...], preferred_element_type=jnp.float32) + b22_ref[...])


_prep = pl.pallas_call(
    _prep_body, out_shape=jax.ShapeDtypeStruct((N, D), jnp.float32))

_layer0 = pl.pallas_call(
    _layer0_body,
    out_shape=(jax.ShapeDtypeStruct((N, D), jnp.float32),
               jax.ShapeDtypeStruct((N, 1), jnp.float32)))

_final = pl.pallas_call(
    _final_body,
    out_shape=(jax.ShapeDtypeStruct((N, D), jnp.float32),
               jax.ShapeDtypeStruct((N, D), jnp.float32)))


def kernel(x, edge_index, W_self0, W_neigh0, b0, W_self1, W_neigh1, b1,
           fc_W, fc_b, bn_gamma, bn_beta, W21, b21, W22, b22):
    src = edge_index[0]
    dst = edge_index[1].reshape(NW, NCH, C)
    zero_block = jnp.zeros((RPW, D), jnp.float32)
    ones_block = jnp.ones((C, D), jnp.float32)

    sc_agg, sc_deg = _sc_calls()
    g = _prep(x)
    degf = sc_deg(dst, zero_block, ones_block)
    agg0 = sc_agg(g, src, dst, zero_block)
    h1, inv = _layer0(g, agg0, degf, W_self0, W_neigh0, b0)
    agg1 = sc_agg(h1, src, dst, zero_block)
    z_loc, z_scale = _final(h1, agg1, inv, W_self1, W_neigh1, b1,
                            fc_W, fc_b, bn_gamma, bn_beta, W21, b21, W22, b22)
    return z_loc, z_scale


# speedup vs baseline: 11.0331x; 1.3268x over previous
import functools

import jax
import jax.numpy as jnp
from jax import lax
from jax.experimental import pallas as pl
from jax.experimental.pallas import tpu as pltpu
from jax.experimental.pallas import tpu_sc as plsc

N = 10000
E = 320000
D = 128

NC = 2
NS = 16
NW = NC * NS
EPW = E // NW
C = 80
NCH = EPW // C
NPAD = 10240
RPW = NPAD // NS


def _sc_agg_kernel(h_hbm, src_hbm, dst_hbm, zero_hbm, agg_hbm,
                   src_pf, dst_vm, rows_v, agg_sh, gsem, ssem, isem):
    cid = lax.axis_index("c")
    sid = lax.axis_index("s")
    wid = cid * NS + sid

    pltpu.sync_copy(dst_hbm.at[wid], dst_vm)
    r0 = pl.multiple_of(sid * RPW, 8)
    pltpu.sync_copy(zero_hbm, agg_sh.at[pl.ds(r0, RPW)])

    base = wid * EPW

    def src_slice(k):
        return src_hbm.at[pl.ds(pl.multiple_of(base + k * C, 8), C)]

    for j in range(3):
        pltpu.async_copy(src_slice(j), src_pf.at[j], isem.at[j])
    for j in range(2):
        pltpu.make_async_copy(src_slice(j), src_pf.at[j], isem.at[j]).wait()
        pltpu.async_copy(h_hbm.at[src_pf.at[j]], rows_v.at[j], gsem.at[j])
    plsc.subcore_barrier()

    def body(k, _):
        b = lax.rem(k, 3)
        b2 = lax.rem(k + 2, 3)
        pltpu.make_async_copy(
            h_hbm.at[src_pf.at[b]], rows_v.at[b], gsem.at[b]).wait()

        @pl.when(k >= 1)
        def _():
            pltpu.make_async_copy(
                rows_v.at[b2], agg_sh.at[dst_vm.at[k - 1]], ssem.at[b2]).wait()

        @pl.when(k + 3 < NCH)
        def _():
            pltpu.async_copy(src_slice(k + 3), src_pf.at[b], isem.at[b])

        @pl.when(k + 2 < NCH)
        def _():
            pltpu.make_async_copy(
                src_slice(k + 2), src_pf.at[b2], isem.at[b2]).wait()
            pltpu.async_copy(h_hbm.at[src_pf.at[b2]], rows_v.at[b2],
                             gsem.at[b2])

        pltpu.async_copy(rows_v.at[b], agg_sh.at[dst_vm.at[k]], ssem.at[b],
                         add=True)
        return 0

    lax.fori_loop(0, NCH, body, 0)
    pltpu.make_async_copy(
        rows_v.at[(NCH - 1) % 3], agg_sh.at[dst_vm.at[NCH - 1]],
        ssem.at[(NCH - 1) % 3]).wait()

    plsc.subcore_barrier()
    pltpu.sync_copy(agg_sh.at[pl.ds(r0, RPW)], agg_hbm.at[cid, pl.ds(r0, RPW)])


def _sc_deg_kernel(dst_hbm, zero_hbm, ones_hbm, deg_hbm,
                   dst_vm, ones_v, deg_sh, ssem):
    cid = lax.axis_index("c")
    sid = lax.axis_index("s")
    wid = cid * NS + sid

    pltpu.sync_copy(dst_hbm.at[wid], dst_vm)
    r0 = pl.multiple_of(sid * RPW, 8)
    pltpu.sync_copy(zero_hbm, deg_sh.at[pl.ds(r0, RPW)])
    pltpu.sync_copy(ones_hbm, ones_v)
    plsc.subcore_barrier()

    def body(k, _):
        @pl.when(k >= 1)
        def _():
            pltpu.make_async_copy(
                ones_v, deg_sh.at[dst_vm.at[k - 1]],
                ssem.at[lax.rem(k - 1, 2)]).wait()

        pltpu.async_copy(ones_v, deg_sh.at[dst_vm.at[k]],
                         ssem.at[lax.rem(k, 2)], add=True)
        return 0

    lax.fori_loop(0, NCH, body, 0)
    pltpu.make_async_copy(
        ones_v, deg_sh.at[dst_vm.at[NCH - 1]], ssem.at[(NCH - 1) % 2]).wait()

    plsc.subcore_barrier()
    pltpu.sync_copy(deg_sh.at[pl.ds(r0, RPW)], deg_hbm.at[cid, pl.ds(r0, RPW)])


@functools.cache
def _sc_calls():
    mesh = plsc.VectorSubcoreMesh(core_axis_name="c", subcore_axis_name="s",
                                  num_cores=NC, num_subcores=NS)
    agg = functools.partial(
        pl.kernel,
        out_type=jax.ShapeDtypeStruct((NC, NPAD, D), jnp.float32),
        mesh=mesh,
        scratch_types=[
            pltpu.VMEM((3, C), jnp.int32),
            pltpu.VMEM((NCH, C), jnp.int32),
            pltpu.VMEM((3, C, D), jnp.float32),
            pltpu.VMEM_SHARED((NPAD, D), jnp.float32),
            pltpu.SemaphoreType.DMA((3,)),
            pltpu.SemaphoreType.DMA((3,)),
            pltpu.SemaphoreType.DMA((3,)),
        ],
    )(_sc_agg_kernel)
    deg = functools.partial(
        pl.kernel,
        out_type=jax.ShapeDtypeStruct((NC, NPAD, D), jnp.float32),
        mesh=mesh,
        scratch_types=[
            pltpu.VMEM((NCH, C), jnp.int32),
            pltpu.VMEM((C, D), jnp.float32),
            pltpu.VMEM_SHARED((NPAD, D), jnp.float32),
            pltpu.SemaphoreType.DMA((2,)),
        ],
    )(_sc_deg_kernel)
    return agg, deg


def _prep_body(x_ref, out_ref):
    out_ref[...] = jnp.log(x_ref[...] + 1.0)


def _layer0_body(g_ref, agg_ref, deg_ref, ws_ref, wn_ref, b_ref,
                 out_ref, inv_ref):
    inv = 1.0 / jnp.maximum(deg_ref[0, :N, 0:1] + deg_ref[1, :N, 0:1], 1.0)
    inv_ref[...] = inv
    hn = (agg_ref[0, :N] + agg_ref[1, :N]) * inv
    h = (jnp.dot(g_ref[...], ws_ref[...], preferred_element_type=jnp.float32)
         + jnp.dot(hn, wn_ref[...], preferred_element_type=jnp.float32)
         + b_ref[...])
    h = jnp.maximum(h, 0.0)
    nrm = jnp.sqrt(jnp.sum(h * h, axis=1, keepdims=True))
    out_ref[...] = h / jnp.maximum(nrm, 1e-12)


def _final_body(h_ref, agg_ref, inv_ref, ws_ref, wn_ref, b_ref, fcw_ref,
                fcb_ref, gam_ref, bet_ref, w21_ref, b21_ref, w22_ref, b22_ref,
                zl_ref, zs_ref):
    hn = (agg_ref[0, :N] + agg_ref[1, :N]) * inv_ref[...]
    h2 = (jnp.dot(h_ref[...], ws_ref[...], preferred_element_type=jnp.float32)
          + jnp.dot(hn, wn_ref[...], preferred_element_type=jnp.float32)
          + b_ref[...])
    t = jnp.dot(h2, fcw_ref[...], preferred_element_type=jnp.float32) + fcb_ref[...]
    mu = jnp.mean(t, axis=0, keepdims=True)
    var = jnp.mean((t - mu) ** 2, axis=0, keepdims=True)
    t = (t - mu) * lax.rsqrt(var + 1e-5) * gam_ref[...] + bet_ref[...]
    t = jnp.maximum(t, 0.0)
    t = jnp.log(1.0 + jnp.exp(-t)) + t
    zl_ref[...] = jnp.dot(t, w21_ref[...], preferred_element_type=jnp.float32) + b21_ref[...]
    zs_ref[...] = jnp.exp(
        jnp.dot(t, w22_ref[...], preferred_element_type=jnp.float32) + b22_ref[...])


_prep = pl.pallas_call(
    _prep_body, out_shape=jax.ShapeDtypeStruct((N, D), jnp.float32))

_layer0 = pl.pallas_call(
    _layer0_body,
    out_shape=(jax.ShapeDtypeStruct((N, D), jnp.float32),
               jax.ShapeDtypeStruct((N, 1), jnp.float32)))

_final = pl.pallas_call(
    _final_body,
    out_shape=(jax.ShapeDtypeStruct((N, D), jnp.float32),
               jax.ShapeDtypeStruct((N, D), jnp.float32)))


def kernel(x, edge_index, W_self0, W_neigh0, b0, W_self1, W_neigh1, b1,
           fc_W, fc_b, bn_gamma, bn_beta, W21, b21, W22, b22):
    src = edge_index[0]
    dst = edge_index[1].reshape(NW, NCH, C)
    zero_block = jnp.zeros((RPW, D), jnp.float32)
    ones_block = jnp.ones((C, D), jnp.float32)

    sc_agg, sc_deg = _sc_calls()
    g = _prep(x)
    degf = sc_deg(dst, zero_block, ones_block)
    agg0 = sc_agg(g, src, dst, zero_block)
    h1, inv = _layer0(g, agg0, degf, W_self0, W_neigh0, b0)
    agg1 = sc_agg(h1, src, dst, zero_block)
    z_loc, z_scale = _final(h1, agg1, inv, W_self1, W_neigh1, b1,
                            fc_W, fc_b, bn_gamma, bn_beta, W21, b21, W22, b22)
    return z_loc, z_scale


# speedup vs baseline: 11.2535x; 1.0200x over previous
import functools

import jax
import jax.numpy as jnp
from jax import lax
from jax.experimental import pallas as pl
from jax.experimental.pallas import tpu as pltpu
from jax.experimental.pallas import tpu_sc as plsc

N = 10000
E = 320000
D = 128

NC = 2
NS = 16
NW = NC * NS
EPW = E // NW
C = 80
NCH = EPW // C
NPAD = 10240
RPW = NPAD // NS


def _sc_agg_kernel(h_hbm, src_hbm, dst_hbm, zero_hbm, agg_hbm,
                   src_pf, dst_vm, rows_v, agg_sh, gsem, ssem, isem):
    cid = lax.axis_index("c")
    sid = lax.axis_index("s")
    wid = cid * NS + sid

    pltpu.sync_copy(dst_hbm.at[wid], dst_vm)
    r0 = pl.multiple_of(sid * RPW, 8)
    pltpu.sync_copy(zero_hbm, agg_sh.at[pl.ds(r0, RPW)])

    base = wid * EPW

    def src_slice(k):
        return src_hbm.at[pl.ds(pl.multiple_of(base + k * C, 8), C)]

    for j in range(3):
        pltpu.async_copy(src_slice(j), src_pf.at[j], isem.at[j])
    for j in range(2):
        pltpu.make_async_copy(src_slice(j), src_pf.at[j], isem.at[j]).wait()
        pltpu.async_copy(h_hbm.at[src_pf.at[j]], rows_v.at[j], gsem.at[j])
    plsc.subcore_barrier()

    def body(k, _):
        b = lax.rem(k, 3)
        b2 = lax.rem(k + 2, 3)
        pltpu.make_async_copy(
            h_hbm.at[src_pf.at[b]], rows_v.at[b], gsem.at[b]).wait()

        @pl.when(k >= 1)
        def _():
            pltpu.make_async_copy(
                rows_v.at[b2], agg_sh.at[dst_vm.at[k - 1]], ssem.at[b2]).wait()

        @pl.when(k + 3 < NCH)
        def _():
            pltpu.async_copy(src_slice(k + 3), src_pf.at[b], isem.at[b])

        @pl.when(k + 2 < NCH)
        def _():
            pltpu.make_async_copy(
                src_slice(k + 2), src_pf.at[b2], isem.at[b2]).wait()
            pltpu.async_copy(h_hbm.at[src_pf.at[b2]], rows_v.at[b2],
                             gsem.at[b2])

        pltpu.async_copy(rows_v.at[b], agg_sh.at[dst_vm.at[k]], ssem.at[b],
                         add=True)
        return 0

    lax.fori_loop(0, NCH, body, 0)
    pltpu.make_async_copy(
        rows_v.at[(NCH - 1) % 3], agg_sh.at[dst_vm.at[NCH - 1]],
        ssem.at[(NCH - 1) % 3]).wait()

    plsc.subcore_barrier()
    pltpu.sync_copy(agg_sh.at[pl.ds(r0, RPW)], agg_hbm.at[cid, pl.ds(r0, RPW)])


def _sc_agg_deg_kernel(h_hbm, src_hbm, dst_hbm, zero_hbm, ones_hbm,
                       agg_hbm, deg_hbm,
                       src_pf, dst_vm, rows_v, agg_sh, gsem, ssem, isem):
    cid = lax.axis_index("c")
    sid = lax.axis_index("s")
    r0 = pl.multiple_of(sid * RPW, 8)
    _sc_agg_kernel(h_hbm, src_hbm, dst_hbm, zero_hbm, agg_hbm,
                   src_pf, dst_vm, rows_v, agg_sh, gsem, ssem, isem)

    pltpu.sync_copy(zero_hbm, agg_sh.at[pl.ds(r0, RPW)])
    pltpu.sync_copy(ones_hbm, rows_v.at[0])
    plsc.subcore_barrier()

    def dbody(k, _):
        @pl.when(k >= 1)
        def _():
            pltpu.make_async_copy(
                rows_v.at[0], agg_sh.at[dst_vm.at[k - 1]],
                ssem.at[lax.rem(k - 1, 2)]).wait()

        pltpu.async_copy(rows_v.at[0], agg_sh.at[dst_vm.at[k]],
                         ssem.at[lax.rem(k, 2)], add=True)
        return 0

    lax.fori_loop(0, NCH, dbody, 0)
    pltpu.make_async_copy(
        rows_v.at[0], agg_sh.at[dst_vm.at[NCH - 1]],
        ssem.at[(NCH - 1) % 2]).wait()

    plsc.subcore_barrier()
    pltpu.sync_copy(agg_sh.at[pl.ds(r0, RPW)], deg_hbm.at[cid, pl.ds(r0, RPW)])


@functools.cache
def _sc_calls():
    mesh = plsc.VectorSubcoreMesh(core_axis_name="c", subcore_axis_name="s",
                                  num_cores=NC, num_subcores=NS)
    agg = functools.partial(
        pl.kernel,
        out_type=jax.ShapeDtypeStruct((NC, NPAD, D), jnp.float32),
        mesh=mesh,
        scratch_types=[
            pltpu.VMEM((3, C), jnp.int32),
            pltpu.VMEM((NCH, C), jnp.int32),
            pltpu.VMEM((3, C, D), jnp.float32),
            pltpu.VMEM_SHARED((NPAD, D), jnp.float32),
            pltpu.SemaphoreType.DMA((3,)),
            pltpu.SemaphoreType.DMA((3,)),
            pltpu.SemaphoreType.DMA((3,)),
        ],
    )(_sc_agg_kernel)
    agg_deg = functools.partial(
        pl.kernel,
        out_type=(jax.ShapeDtypeStruct((NC, NPAD, D), jnp.float32),
                  jax.ShapeDtypeStruct((NC, NPAD, D), jnp.float32)),
        mesh=mesh,
        scratch_types=[
            pltpu.VMEM((3, C), jnp.int32),
            pltpu.VMEM((NCH, C), jnp.int32),
            pltpu.VMEM((3, C, D), jnp.float32),
            pltpu.VMEM_SHARED((NPAD, D), jnp.float32),
            pltpu.SemaphoreType.DMA((3,)),
            pltpu.SemaphoreType.DMA((3,)),
            pltpu.SemaphoreType.DMA((3,)),
        ],
    )(_sc_agg_deg_kernel)
    return agg, agg_deg


def _prep_body(x_ref, out_ref):
    out_ref[...] = jnp.log(x_ref[...] + 1.0)


def _layer0_body(g_ref, agg_ref, deg_ref, ws_ref, wn_ref, b_ref,
                 out_ref, inv_ref):
    inv = 1.0 / jnp.maximum(deg_ref[0, :N, 0:1] + deg_ref[1, :N, 0:1], 1.0)
    inv_ref[...] = inv
    hn = (agg_ref[0, :N] + agg_ref[1, :N]) * inv
    h = (jnp.dot(g_ref[...], ws_ref[...], preferred_element_type=jnp.float32)
         + jnp.dot(hn, wn_ref[...], preferred_element_type=jnp.float32)
         + b_ref[...])
    h = jnp.maximum(h, 0.0)
    nrm = jnp.sqrt(jnp.sum(h * h, axis=1, keepdims=True))
    out_ref[...] = h / jnp.maximum(nrm, 1e-12)


def _final_body(h_ref, agg_ref, inv_ref, ws_ref, wn_ref, b_ref, fcw_ref,
                fcb_ref, gam_ref, bet_ref, w21_ref, b21_ref, w22_ref, b22_ref,
                zl_ref, zs_ref):
    hn = (agg_ref[0, :N] + agg_ref[1, :N]) * inv_ref[...]
    h2 = (jnp.dot(h_ref[...], ws_ref[...], preferred_element_type=jnp.float32)
          + jnp.dot(hn, wn_ref[...], preferred_element_type=jnp.float32)
          + b_ref[...])
    t = jnp.dot(h2, fcw_ref[...], preferred_element_type=jnp.float32) + fcb_ref[...]
    mu = jnp.mean(t, axis=0, keepdims=True)
    var = jnp.mean((t - mu) ** 2, axis=0, keepdims=True)
    t = (t - mu) * lax.rsqrt(var + 1e-5) * gam_ref[...] + bet_ref[...]
    t = jnp.maximum(t, 0.0)
    t = jnp.log(1.0 + jnp.exp(-t)) + t
    zl_ref[...] = jnp.dot(t, w21_ref[...], preferred_element_type=jnp.float32) + b21_ref[...]
    zs_ref[...] = jnp.exp(
        jnp.dot(t, w22_ref[...], preferred_element_type=jnp.float32) + b22_ref[...])


_prep = pl.pallas_call(
    _prep_body, out_shape=jax.ShapeDtypeStruct((N, D), jnp.float32))

_layer0 = pl.pallas_call(
    _layer0_body,
    out_shape=(jax.ShapeDtypeStruct((N, D), jnp.float32),
               jax.ShapeDtypeStruct((N, 1), jnp.float32)))

_final = pl.pallas_call(
    _final_body,
    out_shape=(jax.ShapeDtypeStruct((N, D), jnp.float32),
               jax.ShapeDtypeStruct((N, D), jnp.float32)))


def kernel(x, edge_index, W_self0, W_neigh0, b0, W_self1, W_neigh1, b1,
           fc_W, fc_b, bn_gamma, bn_beta, W21, b21, W22, b22):
    src = edge_index[0]
    dst = edge_index[1].reshape(NW, NCH, C)
    zero_block = jnp.zeros((RPW, D), jnp.float32)
    ones_block = jnp.ones((C, D), jnp.float32)

    sc_agg, sc_agg_deg = _sc_calls()
    g = _prep(x)
    agg0, degf = sc_agg_deg(g, src, dst, zero_block, ones_block)
    h1, inv = _layer0(g, agg0, degf, W_self0, W_neigh0, b0)
    agg1 = sc_agg(h1, src, dst, zero_block)
    z_loc, z_scale = _final(h1, agg1, inv, W_self1, W_neigh1, b1,
                            fc_W, fc_b, bn_gamma, bn_beta, W21, b21, W22, b22)
    return z_loc, z_scale


# speedup vs baseline: 11.9055x; 1.0579x over previous
import functools

import jax
import jax.numpy as jnp
from jax import lax
from jax.experimental import pallas as pl
from jax.experimental.pallas import tpu as pltpu
from jax.experimental.pallas import tpu_sc as plsc

N = 10000
E = 320000
D = 128

NC = 2
NS = 16
NW = NC * NS
EPW = E // NW
C = 80
NCH = EPW // C
NPAD = 10240
RPW = NPAD // NS


def _sc_agg_kernel(h_hbm, src_hbm, dst_hbm, zero_hbm, agg_hbm,
                   src_pf, dst_pf, rows_v, agg_sh, gsem, ssem, isem, jsem):
    cid = lax.axis_index("c")
    sid = lax.axis_index("s")
    wid = cid * NS + sid

    r0 = pl.multiple_of(sid * RPW, 8)
    pltpu.sync_copy(zero_hbm, agg_sh.at[pl.ds(r0, RPW)])

    base = wid * EPW

    def src_slice(k):
        return src_hbm.at[pl.ds(pl.multiple_of(base + k * C, 8), C)]

    def dst_slice(k):
        return dst_hbm.at[pl.ds(pl.multiple_of(base + k * C, 8), C)]

    for j in range(4):
        pltpu.async_copy(src_slice(j), src_pf.at[j], isem.at[j])
        pltpu.async_copy(dst_slice(j), dst_pf.at[j], jsem.at[j])
    for j in range(3):
        pltpu.make_async_copy(src_slice(j), src_pf.at[j], isem.at[j]).wait()
        pltpu.async_copy(h_hbm.at[src_pf.at[j]], rows_v.at[j], gsem.at[j])
    plsc.subcore_barrier()

    def body(k, _):
        b4 = lax.rem(k, 4)
        b6 = lax.rem(k, 6)
        pltpu.make_async_copy(
            h_hbm.at[src_pf.at[b6]], rows_v.at[b4], gsem.at[b4]).wait()

        @pl.when(k >= 1)
        def _():
            pltpu.make_async_copy(
                rows_v.at[lax.rem(k + 3, 4)],
                agg_sh.at[dst_pf.at[lax.rem(k + 5, 6)]],
                ssem.at[lax.rem(k + 3, 4)]).wait()

        @pl.when(k + 4 < NCH)
        def _():
            pltpu.async_copy(src_slice(k + 4), src_pf.at[lax.rem(k + 4, 6)],
                             isem.at[lax.rem(k + 4, 6)])
            pltpu.async_copy(dst_slice(k + 4), dst_pf.at[lax.rem(k + 4, 6)],
                             jsem.at[lax.rem(k + 4, 6)])

        @pl.when(k + 3 < NCH)
        def _():
            pltpu.make_async_copy(
                src_slice(k + 3), src_pf.at[lax.rem(k + 3, 6)],
                isem.at[lax.rem(k + 3, 6)]).wait()
            pltpu.async_copy(h_hbm.at[src_pf.at[lax.rem(k + 3, 6)]],
                             rows_v.at[lax.rem(k + 3, 4)],
                             gsem.at[lax.rem(k + 3, 4)])

        pltpu.make_async_copy(
            dst_slice(k), dst_pf.at[b6], jsem.at[b6]).wait()
        pltpu.async_copy(rows_v.at[b4], agg_sh.at[dst_pf.at[b6]],
                         ssem.at[b4], add=True)
        return 0

    lax.fori_loop(0, NCH, body, 0)
    pltpu.make_async_copy(
        rows_v.at[(NCH - 1) % 4], agg_sh.at[dst_pf.at[(NCH - 1) % 6]],
        ssem.at[(NCH - 1) % 4]).wait()

    plsc.subcore_barrier()
    pltpu.sync_copy(agg_sh.at[pl.ds(r0, RPW)], agg_hbm.at[cid, pl.ds(r0, RPW)])


def _sc_agg_deg_kernel(h_hbm, src_hbm, dst_hbm, zero_hbm, ones_hbm,
                       agg_hbm, deg_hbm,
                       src_pf, dst_pf, rows_v, agg_sh, gsem, ssem, isem, jsem):
    cid = lax.axis_index("c")
    sid = lax.axis_index("s")
    wid = cid * NS + sid
    base = wid * EPW
    r0 = pl.multiple_of(sid * RPW, 8)
    _sc_agg_kernel(h_hbm, src_hbm, dst_hbm, zero_hbm, agg_hbm,
                   src_pf, dst_pf, rows_v, agg_sh, gsem, ssem, isem, jsem)

    pltpu.sync_copy(zero_hbm, agg_sh.at[pl.ds(r0, RPW)])
    pltpu.sync_copy(ones_hbm, rows_v.at[0])

    def dst_slice(k):
        return dst_hbm.at[pl.ds(pl.multiple_of(base + k * C, 8), C)]

    for j in range(3):
        pltpu.async_copy(dst_slice(j), dst_pf.at[j], jsem.at[j])
    plsc.subcore_barrier()

    def dbody(k, _):
        b6 = lax.rem(k, 6)

        @pl.when(k >= 3)
        def _():
            pltpu.make_async_copy(
                rows_v.at[0], agg_sh.at[dst_pf.at[lax.rem(k + 3, 6)]],
                ssem.at[lax.rem(k + 1, 4)]).wait()

        @pl.when(k + 3 < NCH)
        def _():
            pltpu.async_copy(dst_slice(k + 3), dst_pf.at[lax.rem(k + 3, 6)],
                             jsem.at[lax.rem(k + 3, 6)])

        pltpu.make_async_copy(
            dst_slice(k), dst_pf.at[b6], jsem.at[b6]).wait()
        pltpu.async_copy(rows_v.at[0], agg_sh.at[dst_pf.at[b6]],
                         ssem.at[lax.rem(k, 4)], add=True)
        return 0

    lax.fori_loop(0, NCH, dbody, 0)
    for j in range(NCH - 3, NCH):
        pltpu.make_async_copy(
            rows_v.at[0], agg_sh.at[dst_pf.at[j % 6]],
            ssem.at[j % 4]).wait()

    plsc.subcore_barrier()
    pltpu.sync_copy(agg_sh.at[pl.ds(r0, RPW)], deg_hbm.at[cid, pl.ds(r0, RPW)])


@functools.cache
def _sc_calls():
    mesh = plsc.VectorSubcoreMesh(core_axis_name="c", subcore_axis_name="s",
                                  num_cores=NC, num_subcores=NS)
    agg = functools.partial(
        pl.kernel,
        out_type=jax.ShapeDtypeStruct((NC, NPAD, D), jnp.float32),
        mesh=mesh,
        scratch_types=[
            pltpu.VMEM((6, C), jnp.int32),
            pltpu.VMEM((6, C), jnp.int32),
            pltpu.VMEM((4, C, D), jnp.float32),
            pltpu.VMEM_SHARED((NPAD, D), jnp.float32),
            pltpu.SemaphoreType.DMA((4,)),
            pltpu.SemaphoreType.DMA((4,)),
            pltpu.SemaphoreType.DMA((6,)),
            pltpu.SemaphoreType.DMA((6,)),
        ],
    )(_sc_agg_kernel)
    agg_deg = functools.partial(
        pl.kernel,
        out_type=(jax.ShapeDtypeStruct((NC, NPAD, D), jnp.float32),
                  jax.ShapeDtypeStruct((NC, NPAD, D), jnp.float32)),
        mesh=mesh,
        scratch_types=[
            pltpu.VMEM((6, C), jnp.int32),
            pltpu.VMEM((6, C), jnp.int32),
            pltpu.VMEM((4, C, D), jnp.float32),
            pltpu.VMEM_SHARED((NPAD, D), jnp.float32),
            pltpu.SemaphoreType.DMA((4,)),
            pltpu.SemaphoreType.DMA((4,)),
            pltpu.SemaphoreType.DMA((6,)),
            pltpu.SemaphoreType.DMA((6,)),
        ],
    )(_sc_agg_deg_kernel)
    return agg, agg_deg


def _prep_body(x_ref, out_ref):
    out_ref[...] = jnp.log(x_ref[...] + 1.0)


def _layer0_body(g_ref, agg_ref, deg_ref, ws_ref, wn_ref, b_ref,
                 out_ref, inv_ref):
    inv = 1.0 / jnp.maximum(deg_ref[0, :N, 0:1] + deg_ref[1, :N, 0:1], 1.0)
    inv_ref[...] = inv
    hn = (agg_ref[0, :N] + agg_ref[1, :N]) * inv
    h = (jnp.dot(g_ref[...], ws_ref[...], preferred_element_type=jnp.float32)
         + jnp.dot(hn, wn_ref[...], preferred_element_type=jnp.float32)
         + b_ref[...])
    h = jnp.maximum(h, 0.0)
    nrm = jnp.sqrt(jnp.sum(h * h, axis=1, keepdims=True))
    out_ref[...] = h / jnp.maximum(nrm, 1e-12)


def _final_body(h_ref, agg_ref, inv_ref, ws_ref, wn_ref, b_ref, fcw_ref,
                fcb_ref, gam_ref, bet_ref, w21_ref, b21_ref, w22_ref, b22_ref,
                zl_ref, zs_ref):
    hn = (agg_ref[0, :N] + agg_ref[1, :N]) * inv_ref[...]
    h2 = (jnp.dot(h_ref[...], ws_ref[...], preferred_element_type=jnp.float32)
          + jnp.dot(hn, wn_ref[...], preferred_element_type=jnp.float32)
          + b_ref[...])
    t = jnp.dot(h2, fcw_ref[...], preferred_element_type=jnp.float32) + fcb_ref[...]
    mu = jnp.mean(t, axis=0, keepdims=True)
    var = jnp.mean((t - mu) ** 2, axis=0, keepdims=True)
    t = (t - mu) * lax.rsqrt(var + 1e-5) * gam_ref[...] + bet_ref[...]
    t = jnp.maximum(t, 0.0)
    t = jnp.log(1.0 + jnp.exp(-t)) + t
    zl_ref[...] = jnp.dot(t, w21_ref[...], preferred_element_type=jnp.float32) + b21_ref[...]
    zs_ref[...] = jnp.exp(
        jnp.dot(t, w22_ref[...], preferred_element_type=jnp.float32) + b22_ref[...])


_prep = pl.pallas_call(
    _prep_body, out_shape=jax.ShapeDtypeStruct((N, D), jnp.float32))

_layer0 = pl.pallas_call(
    _layer0_body,
    out_shape=(jax.ShapeDtypeStruct((N, D), jnp.float32),
               jax.ShapeDtypeStruct((N, 1), jnp.float32)))

_final = pl.pallas_call(
    _final_body,
    out_shape=(jax.ShapeDtypeStruct((N, D), jnp.float32),
               jax.ShapeDtypeStruct((N, D), jnp.float32)))


def kernel(x, edge_index, W_self0, W_neigh0, b0, W_self1, W_neigh1, b1,
           fc_W, fc_b, bn_gamma, bn_beta, W21, b21, W22, b22):
    src = edge_index[0]
    dst = edge_index[1]
    zero_block = jnp.zeros((RPW, D), jnp.float32)
    ones_block = jnp.ones((C, D), jnp.float32)

    sc_agg, sc_agg_deg = _sc_calls()
    g = _prep(x)
    agg0, degf = sc_agg_deg(g, src, dst, zero_block, ones_block)
    h1, inv = _layer0(g, agg0, degf, W_self0, W_neigh0, b0)
    agg1 = sc_agg(h1, src, dst, zero_block)
    z_loc, z_scale = _final(h1, agg1, inv, W_self1, W_neigh1, b1,
                            fc_W, fc_b, bn_gamma, bn_beta, W21, b21, W22, b22)
    return z_loc, z_scale


# speedup vs baseline: 11.9141x; 1.0007x over previous
import functools

import jax
import jax.numpy as jnp
from jax import lax
from jax.experimental import pallas as pl
from jax.experimental.pallas import tpu as pltpu
from jax.experimental.pallas import tpu_sc as plsc

N = 10000
E = 320000
D = 128

NC = 2
NS = 16
NW = NC * NS
EPW = E // NW
C = 80
NCH = EPW // C
NPAD = 10240
RPW = NPAD // NS


def _sc_agg_kernel(h_hbm, src_hbm, dst_hbm, zero_hbm, agg_hbm,
                   src_pf, dst_pf, rows_v, agg_sh, gsem, ssem, isem, jsem):
    cid = lax.axis_index("c")
    sid = lax.axis_index("s")
    wid = cid * NS + sid

    r0 = pl.multiple_of(sid * RPW, 8)

    base = wid * EPW

    def src_slice(k):
        return src_hbm.at[pl.ds(pl.multiple_of(base + k * C, 8), C)]

    def dst_slice(k):
        return dst_hbm.at[pl.ds(pl.multiple_of(base + k * C, 8), C)]

    for j in range(4):
        pltpu.async_copy(src_slice(j), src_pf.at[j], isem.at[j])
        pltpu.async_copy(dst_slice(j), dst_pf.at[j], jsem.at[j])
    pltpu.sync_copy(zero_hbm, agg_sh.at[pl.ds(r0, RPW)])
    for j in range(3):
        pltpu.make_async_copy(src_slice(j), src_pf.at[j], isem.at[j]).wait()
        pltpu.async_copy(h_hbm.at[src_pf.at[j]], rows_v.at[j], gsem.at[j])
    plsc.subcore_barrier()

    def body(k, _):
        b4 = lax.rem(k, 4)
        b6 = lax.rem(k, 6)
        pltpu.make_async_copy(
            h_hbm.at[src_pf.at[b6]], rows_v.at[b4], gsem.at[b4]).wait()

        @pl.when(k >= 1)
        def _():
            pltpu.make_async_copy(
                rows_v.at[lax.rem(k + 3, 4)],
                agg_sh.at[dst_pf.at[lax.rem(k + 5, 6)]],
                ssem.at[lax.rem(k + 3, 4)]).wait()

        @pl.when(k + 4 < NCH)
        def _():
            pltpu.async_copy(src_slice(k + 4), src_pf.at[lax.rem(k + 4, 6)],
                             isem.at[lax.rem(k + 4, 6)])
            pltpu.async_copy(dst_slice(k + 4), dst_pf.at[lax.rem(k + 4, 6)],
                             jsem.at[lax.rem(k + 4, 6)])

        @pl.when(k + 3 < NCH)
        def _():
            pltpu.make_async_copy(
                src_slice(k + 3), src_pf.at[lax.rem(k + 3, 6)],
                isem.at[lax.rem(k + 3, 6)]).wait()
            pltpu.async_copy(h_hbm.at[src_pf.at[lax.rem(k + 3, 6)]],
                             rows_v.at[lax.rem(k + 3, 4)],
                             gsem.at[lax.rem(k + 3, 4)])

        pltpu.make_async_copy(
            dst_slice(k), dst_pf.at[b6], jsem.at[b6]).wait()
        pltpu.async_copy(rows_v.at[b4], agg_sh.at[dst_pf.at[b6]],
                         ssem.at[b4], add=True)
        return 0

    lax.fori_loop(0, NCH, body, 0)
    pltpu.make_async_copy(
        rows_v.at[(NCH - 1) % 4], agg_sh.at[dst_pf.at[(NCH - 1) % 6]],
        ssem.at[(NCH - 1) % 4]).wait()

    plsc.subcore_barrier()
    pltpu.sync_copy(agg_sh.at[pl.ds(r0, RPW)], agg_hbm.at[cid, pl.ds(r0, RPW)])


def _sc_agg_deg_kernel(h_hbm, src_hbm, dst_hbm, zero_hbm, ones_hbm,
                       agg_hbm, deg_hbm,
                       src_pf, dst_pf, rows_v, agg_sh, gsem, ssem, isem, jsem):
    cid = lax.axis_index("c")
    sid = lax.axis_index("s")
    wid = cid * NS + sid
    base = wid * EPW
    r0 = pl.multiple_of(sid * RPW, 8)
    _sc_agg_kernel(h_hbm, src_hbm, dst_hbm, zero_hbm, agg_hbm,
                   src_pf, dst_pf, rows_v, agg_sh, gsem, ssem, isem, jsem)

    pltpu.sync_copy(zero_hbm, agg_sh.at[pl.ds(r0, RPW)])
    pltpu.sync_copy(ones_hbm, rows_v.at[0])

    def dst_slice(k):
        return dst_hbm.at[pl.ds(pl.multiple_of(base + k * C, 8), C)]

    for j in range(3):
        pltpu.async_copy(dst_slice(j), dst_pf.at[j], jsem.at[j])
    plsc.subcore_barrier()

    def dbody(k, _):
        b6 = lax.rem(k, 6)

        @pl.when(k >= 3)
        def _():
            pltpu.make_async_copy(
                rows_v.at[0], agg_sh.at[dst_pf.at[lax.rem(k + 3, 6)]],
                ssem.at[lax.rem(k + 1, 4)]).wait()

        @pl.when(k + 3 < NCH)
        def _():
            pltpu.async_copy(dst_slice(k + 3), dst_pf.at[lax.rem(k + 3, 6)],
                             jsem.at[lax.rem(k + 3, 6)])

        pltpu.make_async_copy(
            dst_slice(k), dst_pf.at[b6], jsem.at[b6]).wait()
        pltpu.async_copy(rows_v.at[0], agg_sh.at[dst_pf.at[b6]],
                         ssem.at[lax.rem(k, 4)], add=True)
        return 0

    lax.fori_loop(0, NCH, dbody, 0)
    for j in range(NCH - 3, NCH):
        pltpu.make_async_copy(
            rows_v.at[0], agg_sh.at[dst_pf.at[j % 6]],
            ssem.at[j % 4]).wait()

    plsc.subcore_barrier()
    pltpu.sync_copy(agg_sh.at[pl.ds(r0, RPW)], deg_hbm.at[cid, pl.ds(r0, RPW)])


@functools.cache
def _sc_calls():
    mesh = plsc.VectorSubcoreMesh(core_axis_name="c", subcore_axis_name="s",
                                  num_cores=NC, num_subcores=NS)
    agg = functools.partial(
        pl.kernel,
        out_type=jax.ShapeDtypeStruct((NC, NPAD, D), jnp.float32),
        mesh=mesh,
        scratch_types=[
            pltpu.VMEM((6, C), jnp.int32),
            pltpu.VMEM((6, C), jnp.int32),
            pltpu.VMEM((4, C, D), jnp.float32),
            pltpu.VMEM_SHARED((NPAD, D), jnp.float32),
            pltpu.SemaphoreType.DMA((4,)),
            pltpu.SemaphoreType.DMA((4,)),
            pltpu.SemaphoreType.DMA((6,)),
            pltpu.SemaphoreType.DMA((6,)),
        ],
    )(_sc_agg_kernel)
    agg_deg = functools.partial(
        pl.kernel,
        out_type=(jax.ShapeDtypeStruct((NC, NPAD, D), jnp.float32),
                  jax.ShapeDtypeStruct((NC, NPAD, D), jnp.float32)),
        mesh=mesh,
        scratch_types=[
            pltpu.VMEM((6, C), jnp.int32),
            pltpu.VMEM((6, C), jnp.int32),
            pltpu.VMEM((4, C, D), jnp.float32),
            pltpu.VMEM_SHARED((NPAD, D), jnp.float32),
            pltpu.SemaphoreType.DMA((4,)),
            pltpu.SemaphoreType.DMA((4,)),
            pltpu.SemaphoreType.DMA((6,)),
            pltpu.SemaphoreType.DMA((6,)),
        ],
    )(_sc_agg_deg_kernel)
    return agg, agg_deg


def _prep_body(x_ref, out_ref):
    out_ref[...] = jnp.log(x_ref[...] + 1.0)


def _layer0_body(g_ref, agg_ref, deg_ref, ws_ref, wn_ref, b_ref,
                 out_ref, inv_ref):
    inv = 1.0 / jnp.maximum(deg_ref[0, :N, 0:1] + deg_ref[1, :N, 0:1], 1.0)
    inv_ref[...] = inv
    hn = (agg_ref[0, :N] + agg_ref[1, :N]) * inv
    h = (jnp.dot(g_ref[...], ws_ref[...], preferred_element_type=jnp.float32)
         + jnp.dot(hn, wn_ref[...], preferred_element_type=jnp.float32)
         + b_ref[...])
    h = jnp.maximum(h, 0.0)
    nrm = jnp.sqrt(jnp.sum(h * h, axis=1, keepdims=True))
    out_ref[...] = h / jnp.maximum(nrm, 1e-12)


def _final_body(h_ref, agg_ref, inv_ref, ws_ref, wn_ref, b_ref, fcw_ref,
                fcb_ref, gam_ref, bet_ref, w21_ref, b21_ref, w22_ref, b22_ref,
                zl_ref, zs_ref):
    hn = (agg_ref[0, :N] + agg_ref[1, :N]) * inv_ref[...]
    h2 = (jnp.dot(h_ref[...], ws_ref[...], preferred_element_type=jnp.float32)
          + jnp.dot(hn, wn_ref[...], preferred_element_type=jnp.float32)
          + b_ref[...])
    t = jnp.dot(h2, fcw_ref[...], preferred_element_type=jnp.float32) + fcb_ref[...]
    mu = jnp.mean(t, axis=0, keepdims=True)
    var = jnp.mean((t - mu) ** 2, axis=0, keepdims=True)
    t = (t - mu) * lax.rsqrt(var + 1e-5) * gam_ref[...] + bet_ref[...]
    t = jnp.maximum(t, 0.0)
    t = jnp.log(1.0 + jnp.exp(-t)) + t
    zl_ref[...] = jnp.dot(t, w21_ref[...], preferred_element_type=jnp.float32) + b21_ref[...]
    zs_ref[...] = jnp.exp(
        jnp.dot(t, w22_ref[...], preferred_element_type=jnp.float32) + b22_ref[...])


_prep = pl.pallas_call(
    _prep_body, out_shape=jax.ShapeDtypeStruct((N, D), jnp.float32))

_layer0 = pl.pallas_call(
    _layer0_body,
    out_shape=(jax.ShapeDtypeStruct((N, D), jnp.float32),
               jax.ShapeDtypeStruct((N, 1), jnp.float32)))

_final = pl.pallas_call(
    _final_body,
    out_shape=(jax.ShapeDtypeStruct((N, D), jnp.float32),
               jax.ShapeDtypeStruct((N, D), jnp.float32)))


def kernel(x, edge_index, W_self0, W_neigh0, b0, W_self1, W_neigh1, b1,
           fc_W, fc_b, bn_gamma, bn_beta, W21, b21, W22, b22):
    src = edge_index[0]
    dst = edge_index[1]
    zero_block = jnp.zeros((RPW, D), jnp.float32)
    ones_block = jnp.ones((C, D), jnp.float32)

    sc_agg, sc_agg_deg = _sc_calls()
    g = _prep(x)
    agg0, degf = sc_agg_deg(g, src, dst, zero_block, ones_block)
    h1, inv = _layer0(g, agg0, degf, W_self0, W_neigh0, b0)
    agg1 = sc_agg(h1, src, dst, zero_block)
    z_loc, z_scale = _final(h1, agg1, inv, W_self1, W_neigh1, b1,
                            fc_W, fc_b, bn_gamma, bn_beta, W21, b21, W22, b22)
    return z_loc, z_scale
